# Initial kernel scaffold; baseline (speedup 1.0000x reference)
import jax
import jax.numpy as jnp
from jax.experimental import pallas as pl


def kernel(x, edge_index, params):
    raise NotImplementedError("write your pallas kernel here")


# speedup vs baseline: 1.0161x; 1.0161x over previous
import functools

import jax
import jax.numpy as jnp
from jax.experimental import pallas as pl
from jax.experimental.pallas import tpu as pltpu

_N = 10000
_ROWB = 1000
_EPS = 1e-5


def _mm_relu_body(x_ref, w_ref, b_ref, o_ref):
    o_ref[...] = jnp.maximum(
        jnp.dot(x_ref[...], w_ref[...], preferred_element_type=jnp.float32)
        + b_ref[...], 0.0)


def _pool_pre(x, wp, bp):
    n, fi = x.shape
    return pl.pallas_call(
        _mm_relu_body,
        grid=(n // _ROWB,),
        in_specs=[
            pl.BlockSpec((_ROWB, fi), lambda i: (i, 0)),
            pl.BlockSpec((fi, fi), lambda i: (0, 0)),
            pl.BlockSpec((1, fi), lambda i: (0, 0)),
        ],
        out_specs=pl.BlockSpec((_ROWB, fi), lambda i: (i, 0)),
        out_shape=jax.ShapeDtypeStruct((n, fi), jnp.float32),
    )(x, wp, bp.reshape(1, -1))


def _combine_body(stats, x_ref, s_ref, m_ref, a1_ref, a2_ref, wx_ref, wsn_ref,
                  wgn_ref, wpn_ref, b_ref, o_ref, st_ref=None):
    xb = x_ref[...]
    sb = s_ref[...]
    out = jnp.dot(xb, wx_ref[...], preferred_element_type=jnp.float32)
    out += jnp.dot(a1_ref[...] * sb, wsn_ref[...],
                   preferred_element_type=jnp.float32)
    out += jnp.dot(a2_ref[...] * (sb + xb), wgn_ref[...],
                   preferred_element_type=jnp.float32)
    out += jnp.dot(m_ref[...], wpn_ref[...], preferred_element_type=jnp.float32)
    out += b_ref[...]
    if stats:
        o_ref[...] = out

        @pl.when(pl.program_id(0) == 0)
        def _():
            st_ref[...] = jnp.zeros_like(st_ref)

        st_ref[...] += jnp.concatenate(
            [jnp.sum(out, axis=0, keepdims=True),
             jnp.sum(out * out, axis=0, keepdims=True)], axis=0)
    else:
        mx = jnp.max(out, axis=1, keepdims=True)
        lse = jnp.log(jnp.sum(jnp.exp(out - mx), axis=1, keepdims=True)) + mx
        o_ref[...] = out - lse


def _combine(x, s, m, a1, a2, wx, wsn, wgn, wpn, b, stats):
    n, fi = x.shape
    fo = wx.shape[1]
    in_specs = [
        pl.BlockSpec((_ROWB, fi), lambda i: (i, 0)),
        pl.BlockSpec((_ROWB, fi), lambda i: (i, 0)),
        pl.BlockSpec((_ROWB, fi), lambda i: (i, 0)),
        pl.BlockSpec((_ROWB, 1), lambda i: (i, 0)),
        pl.BlockSpec((_ROWB, 1), lambda i: (i, 0)),
        pl.BlockSpec((fi, fo), lambda i: (0, 0)),
        pl.BlockSpec((fi, fo), lambda i: (0, 0)),
        pl.BlockSpec((fi, fo), lambda i: (0, 0)),
        pl.BlockSpec((fi, fo), lambda i: (0, 0)),
        pl.BlockSpec((1, fo), lambda i: (0, 0)),
    ]
    if stats:
        out_specs = [pl.BlockSpec((_ROWB, fo), lambda i: (i, 0)),
                     pl.BlockSpec((2, fo), lambda i: (0, 0))]
        out_shape = [jax.ShapeDtypeStruct((n, fo), jnp.float32),
                     jax.ShapeDtypeStruct((2, fo), jnp.float32)]
    else:
        out_specs = pl.BlockSpec((_ROWB, fo), lambda i: (i, 0))
        out_shape = jax.ShapeDtypeStruct((n, fo), jnp.float32)
    return pl.pallas_call(
        functools.partial(_combine_body, stats),
        grid=(n // _ROWB,),
        in_specs=in_specs,
        out_specs=out_specs,
        out_shape=out_shape,
    )(x, s, m, a1, a2, wx, wsn, wgn, wpn, b.reshape(1, -1))


def _bn_relu_body(h_ref, st_ref, g_ref, b_ref, o_ref):
    s1 = st_ref[0:1, :]
    s2 = st_ref[1:2, :]
    mean = s1 / _N
    var = s2 / _N - mean * mean
    inv = jax.lax.rsqrt(var + _EPS)
    o_ref[...] = jnp.maximum(
        g_ref[...] * (h_ref[...] - mean) * inv + b_ref[...], 0.0)


def _bn_relu(h, st, g, b):
    n, f = h.shape
    return pl.pallas_call(
        _bn_relu_body,
        grid=(n // _ROWB,),
        in_specs=[
            pl.BlockSpec((_ROWB, f), lambda i: (i, 0)),
            pl.BlockSpec((2, f), lambda i: (0, 0)),
            pl.BlockSpec((1, f), lambda i: (0, 0)),
            pl.BlockSpec((1, f), lambda i: (0, 0)),
        ],
        out_specs=pl.BlockSpec((_ROWB, f), lambda i: (i, 0)),
        out_shape=jax.ShapeDtypeStruct((n, f), jnp.float32),
    )(h, st, g.reshape(1, -1), b.reshape(1, -1))


def _seg_sum(rows, dst):
    return jax.ops.segment_sum(rows, dst, num_segments=_N)


def _seg_max0(rows, dst):
    m = jax.ops.segment_max(rows, dst, num_segments=_N)
    return jnp.where(jnp.isfinite(m), m, 0.0)


def _layer(x, src, dst, a1, a2, p, wts, stats):
    s = _seg_sum(x[src], dst)
    hp = _pool_pre(x, p['pool']['Wp'], p['pool']['bp'])
    m = _seg_max0(hp[src], dst)
    w0, w1, w2 = wts[0], wts[1], wts[2]
    wx = w0 * p['mean']['Ws'] + w1 * p['pool']['Ws']
    wsn = w0 * p['mean']['Wn']
    wgn = w2 * p['gcn']['Wn']
    wpn = w1 * p['pool']['Wn']
    b = w0 * p['mean']['b'] + w1 * p['pool']['b'] + w2 * p['gcn']['b']
    return _combine(x, s, m, a1, a2, wx, wsn, wgn, wpn, b, stats)


def kernel(x, edge_index, params):
    src = edge_index[0]
    dst = edge_index[1]
    deg = jax.ops.segment_sum(jnp.ones((src.shape[0],), jnp.float32), dst,
                              num_segments=_N)
    a1 = (1.0 / jnp.maximum(deg, 1.0)).reshape(-1, 1)
    a2 = (1.0 / (deg + 1.0)).reshape(-1, 1)

    w1 = jax.nn.softmax(params['c1']['w'])
    w3 = jax.nn.softmax(params['c3']['w'])
    w5 = jax.nn.softmax(params['c5']['w'])

    h, st = _layer(x, src, dst, a1, a2, params['c1'], w1, True)
    h = _bn_relu(h, st, params['bn2']['g'], params['bn2']['b'])
    h, st = _layer(h, src, dst, a1, a2, params['c3'], w3, True)
    h = _bn_relu(h, st, params['bn4']['g'], params['bn4']['b'])
    return _layer(h, src, dst, a1, a2, params['c5'], w5, False)


# speedup vs baseline: 1.1532x; 1.1349x over previous
import functools

import jax
import jax.numpy as jnp
from jax import lax
from jax.experimental import pallas as pl
from jax.experimental.pallas import tpu as pltpu
from jax.experimental.pallas import tpu_sc as plsc

_N = 10000
_ROWB = 1000
_EPS = 1e-5

_E = 160000
_CHUNK = 128
_CH_PER_SUB = 80
_E_PAD = 16 * _CH_PER_SUB * _CHUNK
_ACC_ROWS = 10112
_ROWS_PER_SUB = 632
_TRASH = 10008


def _segsum_body(halves, x0, x1, srcp, dstp, zeros, out0, out1, src_v, dst_v,
                 rows_v, acc):
    cid = lax.axis_index("c")
    sid = lax.axis_index("s")
    nch = _CH_PER_SUB if halves else _CH_PER_SUB // 2
    idx_off = sid * nch if halves else (sid * 2 + cid) * nch
    pltpu.sync_copy(zeros.at[pl.ds(sid * _ROWS_PER_SUB, _ROWS_PER_SUB)],
                    acc.at[pl.ds(sid * _ROWS_PER_SUB, _ROWS_PER_SUB)])
    pltpu.sync_copy(srcp.at[pl.ds(idx_off, nch)], src_v)
    pltpu.sync_copy(dstp.at[pl.ds(idx_off, nch)], dst_v)
    plsc.subcore_barrier()

    def chunk(j, carry):
        if halves:
            @pl.when(cid == 0)
            def _():
                pltpu.sync_copy(x0.at[src_v.at[j]], rows_v)
                pltpu.sync_copy(rows_v, acc.at[dst_v.at[j]], add=True)

            @pl.when(cid == 1)
            def _():
                pltpu.sync_copy(x1.at[src_v.at[j]], rows_v)
                pltpu.sync_copy(rows_v, acc.at[dst_v.at[j]], add=True)
        else:
            pltpu.sync_copy(x0.at[src_v.at[j]], rows_v)
            pltpu.sync_copy(rows_v, acc.at[dst_v.at[j]], add=True)
        return carry

    lax.fori_loop(0, nch, chunk, 0)
    plsc.subcore_barrier()

    @pl.when(cid == 0)
    def _():
        pltpu.sync_copy(acc.at[pl.ds(sid * _ROWS_PER_SUB, _ROWS_PER_SUB)],
                        out0.at[pl.ds(sid * _ROWS_PER_SUB, _ROWS_PER_SUB)])

    @pl.when(cid == 1)
    def _():
        pltpu.sync_copy(acc.at[pl.ds(sid * _ROWS_PER_SUB, _ROWS_PER_SUB)],
                        out1.at[pl.ds(sid * _ROWS_PER_SUB, _ROWS_PER_SUB)])


@functools.lru_cache(maxsize=None)
def _segsum_call(halves):
    nch = _CH_PER_SUB if halves else _CH_PER_SUB // 2
    return pl.kernel(
        functools.partial(_segsum_body, halves),
        out_type=[jax.ShapeDtypeStruct((_ACC_ROWS, 128), jnp.float32),
                  jax.ShapeDtypeStruct((_ACC_ROWS, 128), jnp.float32)],
        mesh=plsc.VectorSubcoreMesh(core_axis_name="c", subcore_axis_name="s"),
        scratch_types=[
            pltpu.VMEM((nch, _CHUNK), jnp.int32),
            pltpu.VMEM((nch, _CHUNK), jnp.int32),
            pltpu.VMEM((_CHUNK, 128), jnp.float32),
            pltpu.VMEM_SHARED((_ACC_ROWS, 128), jnp.float32),
        ],
    )


def _sc_segsum(x, srcp, dstp):
    halves = x.shape[1] == 256
    z = jnp.zeros((_ACC_ROWS, 128), jnp.float32)
    if halves:
        out0, out1 = _segsum_call(True)(x[:, :128], x[:, 128:], srcp, dstp, z)
    else:
        out0, out1 = _segsum_call(False)(x, x, srcp, dstp, z)
    return out0, out1, halves


def _mm_relu_body(x_ref, w_ref, b_ref, o_ref):
    o_ref[...] = jnp.maximum(
        jnp.dot(x_ref[...], w_ref[...], preferred_element_type=jnp.float32)
        + b_ref[...], 0.0)


def _pool_pre(x, wp, bp):
    n, fi = x.shape
    return pl.pallas_call(
        _mm_relu_body,
        grid=(n // _ROWB,),
        in_specs=[
            pl.BlockSpec((_ROWB, fi), lambda i: (i, 0)),
            pl.BlockSpec((fi, fi), lambda i: (0, 0)),
            pl.BlockSpec((1, fi), lambda i: (0, 0)),
        ],
        out_specs=pl.BlockSpec((_ROWB, fi), lambda i: (i, 0)),
        out_shape=jax.ShapeDtypeStruct((n, fi), jnp.float32),
    )(x, wp, bp.reshape(1, -1))


def _combine_body(stats, halves, x_ref, s0_ref, s1_ref, m_ref, a1_ref, a2_ref,
                  wx_ref, wsn_ref, wgn_ref, wpn_ref, b_ref, o_ref,
                  st_ref=None):
    xb = x_ref[...]
    if halves:
        sb = jnp.concatenate([s0_ref[...], s1_ref[...]], axis=1)
    else:
        sb = s0_ref[...] + s1_ref[...]
    out = jnp.dot(xb, wx_ref[...], preferred_element_type=jnp.float32)
    out += jnp.dot(a1_ref[...] * sb, wsn_ref[...],
                   preferred_element_type=jnp.float32)
    out += jnp.dot(a2_ref[...] * (sb + xb), wgn_ref[...],
                   preferred_element_type=jnp.float32)
    out += jnp.dot(m_ref[...], wpn_ref[...], preferred_element_type=jnp.float32)
    out += b_ref[...]
    if stats:
        o_ref[...] = out

        @pl.when(pl.program_id(0) == 0)
        def _():
            st_ref[...] = jnp.zeros_like(st_ref)

        st_ref[...] += jnp.concatenate(
            [jnp.sum(out, axis=0, keepdims=True),
             jnp.sum(out * out, axis=0, keepdims=True)], axis=0)
    else:
        mx = jnp.max(out, axis=1, keepdims=True)
        lse = jnp.log(jnp.sum(jnp.exp(out - mx), axis=1, keepdims=True)) + mx
        o_ref[...] = out - lse


def _combine(x, s0, s1, halves, m, a1, a2, wx, wsn, wgn, wpn, b, stats):
    n, fi = x.shape
    fo = wx.shape[1]
    in_specs = [
        pl.BlockSpec((_ROWB, fi), lambda i: (i, 0)),
        pl.BlockSpec((_ROWB, 128), lambda i: (i, 0)),
        pl.BlockSpec((_ROWB, 128), lambda i: (i, 0)),
        pl.BlockSpec((_ROWB, fi), lambda i: (i, 0)),
        pl.BlockSpec((_ROWB, 1), lambda i: (i, 0)),
        pl.BlockSpec((_ROWB, 1), lambda i: (i, 0)),
        pl.BlockSpec((fi, fo), lambda i: (0, 0)),
        pl.BlockSpec((fi, fo), lambda i: (0, 0)),
        pl.BlockSpec((fi, fo), lambda i: (0, 0)),
        pl.BlockSpec((fi, fo), lambda i: (0, 0)),
        pl.BlockSpec((1, fo), lambda i: (0, 0)),
    ]
    if stats:
        out_specs = [pl.BlockSpec((_ROWB, fo), lambda i: (i, 0)),
                     pl.BlockSpec((2, fo), lambda i: (0, 0))]
        out_shape = [jax.ShapeDtypeStruct((n, fo), jnp.float32),
                     jax.ShapeDtypeStruct((2, fo), jnp.float32)]
    else:
        out_specs = pl.BlockSpec((_ROWB, fo), lambda i: (i, 0))
        out_shape = jax.ShapeDtypeStruct((n, fo), jnp.float32)
    return pl.pallas_call(
        functools.partial(_combine_body, stats, halves),
        grid=(n // _ROWB,),
        in_specs=in_specs,
        out_specs=out_specs,
        out_shape=out_shape,
    )(x, s0, s1, m, a1, a2, wx, wsn, wgn, wpn, b.reshape(1, -1))


def _bn_relu_body(h_ref, st_ref, g_ref, b_ref, o_ref):
    s1 = st_ref[0:1, :]
    s2 = st_ref[1:2, :]
    mean = s1 / _N
    var = s2 / _N - mean * mean
    inv = jax.lax.rsqrt(var + _EPS)
    o_ref[...] = jnp.maximum(
        g_ref[...] * (h_ref[...] - mean) * inv + b_ref[...], 0.0)


def _bn_relu(h, st, g, b):
    n, f = h.shape
    return pl.pallas_call(
        _bn_relu_body,
        grid=(n // _ROWB,),
        in_specs=[
            pl.BlockSpec((_ROWB, f), lambda i: (i, 0)),
            pl.BlockSpec((2, f), lambda i: (0, 0)),
            pl.BlockSpec((1, f), lambda i: (0, 0)),
            pl.BlockSpec((1, f), lambda i: (0, 0)),
        ],
        out_specs=pl.BlockSpec((_ROWB, f), lambda i: (i, 0)),
        out_shape=jax.ShapeDtypeStruct((n, f), jnp.float32),
    )(h, st, g.reshape(1, -1), b.reshape(1, -1))


def _seg_max0(rows, dst):
    m = jax.ops.segment_max(rows, dst, num_segments=_N)
    return jnp.where(jnp.isfinite(m), m, 0.0)


def _layer(x, src, dst, srcp, dstp, a1, a2, p, wts, stats):
    s0, s1, halves = _sc_segsum(x, srcp, dstp)
    hp = _pool_pre(x, p['pool']['Wp'], p['pool']['bp'])
    m = _seg_max0(hp[src], dst)
    w0, w1, w2 = wts[0], wts[1], wts[2]
    wx = w0 * p['mean']['Ws'] + w1 * p['pool']['Ws']
    wsn = w0 * p['mean']['Wn']
    wgn = w2 * p['gcn']['Wn']
    wpn = w1 * p['pool']['Wn']
    b = w0 * p['mean']['b'] + w1 * p['pool']['b'] + w2 * p['gcn']['b']
    return _combine(x, s0, s1, halves, m, a1, a2, wx, wsn, wgn, wpn, b, stats)


def kernel(x, edge_index, params):
    src = edge_index[0]
    dst = edge_index[1]
    srcp = jnp.concatenate(
        [src, jnp.zeros((_E_PAD - _E,), jnp.int32)]).reshape(-1, _CHUNK)
    dstp = jnp.concatenate(
        [dst, jnp.full((_E_PAD - _E,), _TRASH, jnp.int32)]).reshape(-1, _CHUNK)
    deg = jax.ops.segment_sum(jnp.ones((src.shape[0],), jnp.float32), dst,
                              num_segments=_N)
    a1 = (1.0 / jnp.maximum(deg, 1.0)).reshape(-1, 1)
    a2 = (1.0 / (deg + 1.0)).reshape(-1, 1)

    w1 = jax.nn.softmax(params['c1']['w'])
    w3 = jax.nn.softmax(params['c3']['w'])
    w5 = jax.nn.softmax(params['c5']['w'])

    h, st = _layer(x, src, dst, srcp, dstp, a1, a2, params['c1'], w1, True)
    h = _bn_relu(h, st, params['bn2']['g'], params['bn2']['b'])
    h, st = _layer(h, src, dst, srcp, dstp, a1, a2, params['c3'], w3, True)
    h = _bn_relu(h, st, params['bn4']['g'], params['bn4']['b'])
    return _layer(h, src, dst, srcp, dstp, a1, a2, params['c5'], w5, False)


# speedup vs baseline: 1.5306x; 1.3273x over previous
import functools

import jax
import jax.numpy as jnp
from jax import lax
from jax.experimental import pallas as pl
from jax.experimental.pallas import tpu as pltpu
from jax.experimental.pallas import tpu_sc as plsc

_N = 10000
_ROWB = 1000
_EPS = 1e-5

_E = 160000
_CHUNK = 128
_CH_PER_SUB = 80
_E_PAD = 16 * _CH_PER_SUB * _CHUNK
_ACC_ROWS = 10112
_ROWS_PER_SUB = 632
_TRASH = 10008


def _segsum_body(halves, x0, x1, srcp, dstp, zeros, out0, out1, src_v, dst_v,
                 rows_v, acc):
    cid = lax.axis_index("c")
    sid = lax.axis_index("s")
    nch = _CH_PER_SUB if halves else _CH_PER_SUB // 2
    idx_off = sid * nch if halves else (sid * 2 + cid) * nch
    pltpu.sync_copy(zeros.at[pl.ds(sid * _ROWS_PER_SUB, _ROWS_PER_SUB)],
                    acc.at[pl.ds(sid * _ROWS_PER_SUB, _ROWS_PER_SUB)])
    pltpu.sync_copy(srcp.at[pl.ds(idx_off, nch)], src_v)
    pltpu.sync_copy(dstp.at[pl.ds(idx_off, nch)], dst_v)
    plsc.subcore_barrier()

    def chunk(j, carry):
        if halves:
            @pl.when(cid == 0)
            def _():
                pltpu.sync_copy(x0.at[src_v.at[j]], rows_v)
                pltpu.sync_copy(rows_v, acc.at[dst_v.at[j]], add=True)

            @pl.when(cid == 1)
            def _():
                pltpu.sync_copy(x1.at[src_v.at[j]], rows_v)
                pltpu.sync_copy(rows_v, acc.at[dst_v.at[j]], add=True)
        else:
            pltpu.sync_copy(x0.at[src_v.at[j]], rows_v)
            pltpu.sync_copy(rows_v, acc.at[dst_v.at[j]], add=True)
        return carry

    lax.fori_loop(0, nch, chunk, 0)
    plsc.subcore_barrier()

    @pl.when(cid == 0)
    def _():
        pltpu.sync_copy(acc.at[pl.ds(sid * _ROWS_PER_SUB, _ROWS_PER_SUB)],
                        out0.at[pl.ds(sid * _ROWS_PER_SUB, _ROWS_PER_SUB)])

    @pl.when(cid == 1)
    def _():
        pltpu.sync_copy(acc.at[pl.ds(sid * _ROWS_PER_SUB, _ROWS_PER_SUB)],
                        out1.at[pl.ds(sid * _ROWS_PER_SUB, _ROWS_PER_SUB)])


@functools.lru_cache(maxsize=None)
def _segsum_call(halves):
    nch = _CH_PER_SUB if halves else _CH_PER_SUB // 2
    return pl.kernel(
        functools.partial(_segsum_body, halves),
        out_type=[jax.ShapeDtypeStruct((_ACC_ROWS, 128), jnp.float32),
                  jax.ShapeDtypeStruct((_ACC_ROWS, 128), jnp.float32)],
        mesh=plsc.VectorSubcoreMesh(core_axis_name="c", subcore_axis_name="s"),
        scratch_types=[
            pltpu.VMEM((nch, _CHUNK), jnp.int32),
            pltpu.VMEM((nch, _CHUNK), jnp.int32),
            pltpu.VMEM((_CHUNK, 128), jnp.float32),
            pltpu.VMEM_SHARED((_ACC_ROWS, 128), jnp.float32),
        ],
    )


_GROUPS = 157
_ES_PAD = _GROUPS * 1024
_NODES_PER_SUB = 320
_NODE_ROWS = 10240


def _segmax_body(f, hp, srcg, dstf, offs_h, out, srcb, drow, rows_v, offs_v,
                 acc):
    cid = lax.axis_index("c")
    sid = lax.axis_index("s")
    w = cid * 16 + sid

    def zero(i, c):
        acc[pl.ds(i * 16, 16)] = jnp.zeros((16,), jnp.float32)
        return c

    lax.fori_loop(0, _NODES_PER_SUB * f // 16, zero, 0)
    pltpu.sync_copy(offs_h, offs_v)
    ob = offs_v[pl.ds(w, 16)]
    s = ob[0]
    e_end = ob[1]
    node_lo = w * _NODES_PER_SUB
    g0 = s // 1024
    g1 = (e_end + 1023) // 1024

    def group(g, c):
        pltpu.sync_copy(srcg.at[pl.ds(g * 8, 8)], srcb)

        def subchunk(j, c2):
            base = g * 1024 + j * 128

            @pl.when((base + _CHUNK > s) & (base < e_end))
            def _():
                pltpu.sync_copy(dstf.at[pl.ds(base, _CHUNK)],
                                drow.at[pl.ds(0, _CHUNK)])
                pltpu.sync_copy(hp.at[srcb.at[j]], rows_v)

                def lane(e, c3):
                    ge = base + e

                    @pl.when((ge >= s) & (ge < e_end))
                    def _():
                        d = drow[pl.ds(e, 16)][0]
                        ao = (d - node_lo) * f
                        for jj in range(f // 16):
                            a = acc[pl.ds(ao + jj * 16, 16)]
                            r = rows_v[e, pl.ds(jj * 16, 16)]
                            acc[pl.ds(ao + jj * 16, 16)] = jnp.maximum(a, r)

                    return c3

                lax.fori_loop(0, _CHUNK, lane, 0)

            return c2

        lax.fori_loop(0, 8, subchunk, c)
        return c

    lax.fori_loop(g0, g1, group, 0)
    pltpu.sync_copy(
        acc, out.at[pl.ds(node_lo * f, _NODES_PER_SUB * f)])


@functools.lru_cache(maxsize=None)
def _segmax_call(f):
    return pl.kernel(
        functools.partial(_segmax_body, f),
        out_type=jax.ShapeDtypeStruct((_NODE_ROWS * f,), jnp.float32),
        mesh=plsc.VectorSubcoreMesh(core_axis_name="c", subcore_axis_name="s"),
        scratch_types=[
            pltpu.VMEM((8, _CHUNK), jnp.int32),
            pltpu.VMEM((_CHUNK + 16,), jnp.int32),
            pltpu.VMEM((_CHUNK, f), jnp.float32),
            pltpu.VMEM((48,), jnp.int32),
            pltpu.VMEM((_NODES_PER_SUB * f,), jnp.float32),
        ],
    )


def _sc_segmax(hp, srcg, dstf, offs48):
    f = hp.shape[1]
    out = _segmax_call(f)(hp, srcg, dstf, offs48)
    return out.reshape(_NODE_ROWS, f)[:_N]


def _sc_segsum(x, srcp, dstp):
    halves = x.shape[1] == 256
    z = jnp.zeros((_ACC_ROWS, 128), jnp.float32)
    if halves:
        out0, out1 = _segsum_call(True)(x[:, :128], x[:, 128:], srcp, dstp, z)
    else:
        out0, out1 = _segsum_call(False)(x, x, srcp, dstp, z)
    return out0, out1, halves


def _mm_relu_body(x_ref, w_ref, b_ref, o_ref):
    o_ref[...] = jnp.maximum(
        jnp.dot(x_ref[...], w_ref[...], preferred_element_type=jnp.float32)
        + b_ref[...], 0.0)


def _pool_pre(x, wp, bp):
    n, fi = x.shape
    return pl.pallas_call(
        _mm_relu_body,
        grid=(n // _ROWB,),
        in_specs=[
            pl.BlockSpec((_ROWB, fi), lambda i: (i, 0)),
            pl.BlockSpec((fi, fi), lambda i: (0, 0)),
            pl.BlockSpec((1, fi), lambda i: (0, 0)),
        ],
        out_specs=pl.BlockSpec((_ROWB, fi), lambda i: (i, 0)),
        out_shape=jax.ShapeDtypeStruct((n, fi), jnp.float32),
    )(x, wp, bp.reshape(1, -1))


def _combine_body(stats, halves, x_ref, s0_ref, s1_ref, m_ref, a1_ref, a2_ref,
                  wx_ref, wsn_ref, wgn_ref, wpn_ref, b_ref, o_ref,
                  st_ref=None):
    xb = x_ref[...]
    if halves:
        sb = jnp.concatenate([s0_ref[...], s1_ref[...]], axis=1)
    else:
        sb = s0_ref[...] + s1_ref[...]
    out = jnp.dot(xb, wx_ref[...], preferred_element_type=jnp.float32)
    out += jnp.dot(a1_ref[...] * sb, wsn_ref[...],
                   preferred_element_type=jnp.float32)
    out += jnp.dot(a2_ref[...] * (sb + xb), wgn_ref[...],
                   preferred_element_type=jnp.float32)
    out += jnp.dot(m_ref[...], wpn_ref[...], preferred_element_type=jnp.float32)
    out += b_ref[...]
    if stats:
        o_ref[...] = out

        @pl.when(pl.program_id(0) == 0)
        def _():
            st_ref[...] = jnp.zeros_like(st_ref)

        st_ref[...] += jnp.concatenate(
            [jnp.sum(out, axis=0, keepdims=True),
             jnp.sum(out * out, axis=0, keepdims=True)], axis=0)
    else:
        mx = jnp.max(out, axis=1, keepdims=True)
        lse = jnp.log(jnp.sum(jnp.exp(out - mx), axis=1, keepdims=True)) + mx
        o_ref[...] = out - lse


def _combine(x, s0, s1, halves, m, a1, a2, wx, wsn, wgn, wpn, b, stats):
    n, fi = x.shape
    fo = wx.shape[1]
    in_specs = [
        pl.BlockSpec((_ROWB, fi), lambda i: (i, 0)),
        pl.BlockSpec((_ROWB, 128), lambda i: (i, 0)),
        pl.BlockSpec((_ROWB, 128), lambda i: (i, 0)),
        pl.BlockSpec((_ROWB, fi), lambda i: (i, 0)),
        pl.BlockSpec((_ROWB, 1), lambda i: (i, 0)),
        pl.BlockSpec((_ROWB, 1), lambda i: (i, 0)),
        pl.BlockSpec((fi, fo), lambda i: (0, 0)),
        pl.BlockSpec((fi, fo), lambda i: (0, 0)),
        pl.BlockSpec((fi, fo), lambda i: (0, 0)),
        pl.BlockSpec((fi, fo), lambda i: (0, 0)),
        pl.BlockSpec((1, fo), lambda i: (0, 0)),
    ]
    if stats:
        out_specs = [pl.BlockSpec((_ROWB, fo), lambda i: (i, 0)),
                     pl.BlockSpec((2, fo), lambda i: (0, 0))]
        out_shape = [jax.ShapeDtypeStruct((n, fo), jnp.float32),
                     jax.ShapeDtypeStruct((2, fo), jnp.float32)]
    else:
        out_specs = pl.BlockSpec((_ROWB, fo), lambda i: (i, 0))
        out_shape = jax.ShapeDtypeStruct((n, fo), jnp.float32)
    return pl.pallas_call(
        functools.partial(_combine_body, stats, halves),
        grid=(n // _ROWB,),
        in_specs=in_specs,
        out_specs=out_specs,
        out_shape=out_shape,
    )(x, s0, s1, m, a1, a2, wx, wsn, wgn, wpn, b.reshape(1, -1))


def _bn_relu_body(h_ref, st_ref, g_ref, b_ref, o_ref):
    s1 = st_ref[0:1, :]
    s2 = st_ref[1:2, :]
    mean = s1 / _N
    var = s2 / _N - mean * mean
    inv = jax.lax.rsqrt(var + _EPS)
    o_ref[...] = jnp.maximum(
        g_ref[...] * (h_ref[...] - mean) * inv + b_ref[...], 0.0)


def _bn_relu(h, st, g, b):
    n, f = h.shape
    return pl.pallas_call(
        _bn_relu_body,
        grid=(n // _ROWB,),
        in_specs=[
            pl.BlockSpec((_ROWB, f), lambda i: (i, 0)),
            pl.BlockSpec((2, f), lambda i: (0, 0)),
            pl.BlockSpec((1, f), lambda i: (0, 0)),
            pl.BlockSpec((1, f), lambda i: (0, 0)),
        ],
        out_specs=pl.BlockSpec((_ROWB, f), lambda i: (i, 0)),
        out_shape=jax.ShapeDtypeStruct((n, f), jnp.float32),
    )(h, st, g.reshape(1, -1), b.reshape(1, -1))


def _layer(x, srcp, dstp, srcg, dstf, offs48, a1, a2, p, wts, stats):
    s0, s1, halves = _sc_segsum(x, srcp, dstp)
    hp = _pool_pre(x, p['pool']['Wp'], p['pool']['bp'])
    m = _sc_segmax(hp, srcg, dstf, offs48)
    w0, w1, w2 = wts[0], wts[1], wts[2]
    wx = w0 * p['mean']['Ws'] + w1 * p['pool']['Ws']
    wsn = w0 * p['mean']['Wn']
    wgn = w2 * p['gcn']['Wn']
    wpn = w1 * p['pool']['Wn']
    b = w0 * p['mean']['b'] + w1 * p['pool']['b'] + w2 * p['gcn']['b']
    return _combine(x, s0, s1, halves, m, a1, a2, wx, wsn, wgn, wpn, b, stats)


def kernel(x, edge_index, params):
    src = edge_index[0]
    dst = edge_index[1]
    srcp = jnp.concatenate(
        [src, jnp.zeros((_E_PAD - _E,), jnp.int32)]).reshape(-1, _CHUNK)
    dstp = jnp.concatenate(
        [dst, jnp.full((_E_PAD - _E,), _TRASH, jnp.int32)]).reshape(-1, _CHUNK)

    dsts, srcs = lax.sort_key_val(dst, src)
    pad = _ES_PAD - _E
    dstf = jnp.concatenate(
        [dsts, jnp.full((pad,), jnp.int32(0x3FFFFFFF))])
    srcg = jnp.concatenate([srcs, jnp.zeros((pad,), jnp.int32)]
                           ).reshape(-1, _CHUNK)
    bounds = jnp.arange(0, _NODE_ROWS + 1, _NODES_PER_SUB, dtype=jnp.int32)
    offs33 = jnp.searchsorted(dsts, bounds).astype(jnp.int32)
    offs48 = jnp.concatenate([offs33, jnp.zeros((15,), jnp.int32)])
    offsn = jnp.searchsorted(dsts, jnp.arange(_N + 1, dtype=jnp.int32))
    deg = (offsn[1:] - offsn[:-1]).astype(jnp.float32)
    a1 = (1.0 / jnp.maximum(deg, 1.0)).reshape(-1, 1)
    a2 = (1.0 / (deg + 1.0)).reshape(-1, 1)

    w1 = jax.nn.softmax(params['c1']['w'])
    w3 = jax.nn.softmax(params['c3']['w'])
    w5 = jax.nn.softmax(params['c5']['w'])

    h, st = _layer(x, srcp, dstp, srcg, dstf, offs48, a1, a2,
                   params['c1'], w1, True)
    h = _bn_relu(h, st, params['bn2']['g'], params['bn2']['b'])
    h, st = _layer(h, srcp, dstp, srcg, dstf, offs48, a1, a2,
                   params['c3'], w3, True)
    h = _bn_relu(h, st, params['bn4']['g'], params['bn4']['b'])
    return _layer(h, srcp, dstp, srcg, dstf, offs48, a1, a2,
                  params['c5'], w5, False)


# speedup vs baseline: 1.5832x; 1.0344x over previous
import functools

import jax
import jax.numpy as jnp
from jax import lax
from jax.experimental import pallas as pl
from jax.experimental.pallas import tpu as pltpu
from jax.experimental.pallas import tpu_sc as plsc

_N = 10000
_ROWB = 1000
_EPS = 1e-5

_E = 160000
_CHUNK = 128
_CH_PER_SUB = 80
_E_PAD = 16 * _CH_PER_SUB * _CHUNK
_ACC_ROWS = 10112
_ROWS_PER_SUB = 632
_TRASH = 10008


def _segsum_body(halves, x0, x1, srcp, dstp, zeros, out0, out1, src_v, dst_v,
                 rows_v, acc):
    cid = lax.axis_index("c")
    sid = lax.axis_index("s")
    nch = _CH_PER_SUB if halves else _CH_PER_SUB // 2
    idx_off = sid * nch if halves else (sid * 2 + cid) * nch
    pltpu.sync_copy(zeros.at[pl.ds(sid * _ROWS_PER_SUB, _ROWS_PER_SUB)],
                    acc.at[pl.ds(sid * _ROWS_PER_SUB, _ROWS_PER_SUB)])
    pltpu.sync_copy(srcp.at[pl.ds(idx_off, nch)], src_v)
    pltpu.sync_copy(dstp.at[pl.ds(idx_off, nch)], dst_v)
    plsc.subcore_barrier()

    def chunk(j, carry):
        if halves:
            @pl.when(cid == 0)
            def _():
                pltpu.sync_copy(x0.at[src_v.at[j]], rows_v)
                pltpu.sync_copy(rows_v, acc.at[dst_v.at[j]], add=True)

            @pl.when(cid == 1)
            def _():
                pltpu.sync_copy(x1.at[src_v.at[j]], rows_v)
                pltpu.sync_copy(rows_v, acc.at[dst_v.at[j]], add=True)
        else:
            pltpu.sync_copy(x0.at[src_v.at[j]], rows_v)
            pltpu.sync_copy(rows_v, acc.at[dst_v.at[j]], add=True)
        return carry

    lax.fori_loop(0, nch, chunk, 0)
    plsc.subcore_barrier()

    @pl.when(cid == 0)
    def _():
        pltpu.sync_copy(acc.at[pl.ds(sid * _ROWS_PER_SUB, _ROWS_PER_SUB)],
                        out0.at[pl.ds(sid * _ROWS_PER_SUB, _ROWS_PER_SUB)])

    @pl.when(cid == 1)
    def _():
        pltpu.sync_copy(acc.at[pl.ds(sid * _ROWS_PER_SUB, _ROWS_PER_SUB)],
                        out1.at[pl.ds(sid * _ROWS_PER_SUB, _ROWS_PER_SUB)])


@functools.lru_cache(maxsize=None)
def _segsum_call(halves):
    nch = _CH_PER_SUB if halves else _CH_PER_SUB // 2
    return pl.kernel(
        functools.partial(_segsum_body, halves),
        out_type=[jax.ShapeDtypeStruct((_ACC_ROWS, 128), jnp.float32),
                  jax.ShapeDtypeStruct((_ACC_ROWS, 128), jnp.float32)],
        mesh=plsc.VectorSubcoreMesh(core_axis_name="c", subcore_axis_name="s"),
        scratch_types=[
            pltpu.VMEM((nch, _CHUNK), jnp.int32),
            pltpu.VMEM((nch, _CHUNK), jnp.int32),
            pltpu.VMEM((_CHUNK, 128), jnp.float32),
            pltpu.VMEM_SHARED((_ACC_ROWS, 128), jnp.float32),
        ],
    )


_MCH = 64
_MGROUP = 8 * _MCH
_GROUPS = 313
_ES_PAD = _GROUPS * _MGROUP
_NODES_PER_SUB = 320
_NODE_ROWS = 10240


def _segmax_body(f, hp, srcg, dstf, offs_h, out, srcb, drow, rows_a, rows_b,
                 offs_v, acc, sem_a, sem_b):
    cid = lax.axis_index("c")
    sid = lax.axis_index("s")
    w = cid * 16 + sid

    def zero(i, c):
        acc[pl.ds(i * 16, 16)] = jnp.zeros((16,), jnp.float32)
        return c

    lax.fori_loop(0, _NODES_PER_SUB * f // 16, zero, 0)
    pltpu.sync_copy(offs_h, offs_v)
    ob = offs_v[pl.ds(w, 16)]
    s = ob[0]
    e_end = ob[1]
    node_lo = w * _NODES_PER_SUB
    g0 = s // _MGROUP
    g1 = (e_end + _MGROUP - 1) // _MGROUP

    def process(base, rows):
        @pl.when((base + _MCH > s) & (base < e_end))
        def _():
            pltpu.sync_copy(dstf.at[pl.ds(base, _MCH)],
                            drow.at[pl.ds(0, _MCH)])

            def lane(e, c3):
                ge = base + e

                @pl.when((ge >= s) & (ge < e_end))
                def _():
                    d = drow[pl.ds(e, 16)][0]
                    ao = (d - node_lo) * f
                    for jj in range(f // 16):
                        a = acc[pl.ds(ao + jj * 16, 16)]
                        r = rows[e, pl.ds(jj * 16, 16)]
                        acc[pl.ds(ao + jj * 16, 16)] = jnp.maximum(a, r)

                return c3

            lax.fori_loop(0, _MCH, lane, 0)

    def group(g, c):
        pltpu.sync_copy(srcg.at[pl.ds(g * 8, 8)], srcb)
        pltpu.make_async_copy(hp.at[srcb.at[0]], rows_a, sem_a).start()

        def pair(j2, c2):
            j = j2 * 2
            base = g * _MGROUP + j * _MCH
            pltpu.make_async_copy(hp.at[srcb.at[j]], rows_a, sem_a).wait()
            pltpu.make_async_copy(hp.at[srcb.at[j + 1]], rows_b, sem_b).start()
            process(base, rows_a)
            pltpu.make_async_copy(hp.at[srcb.at[j + 1]], rows_b, sem_b).wait()

            @pl.when(j + 2 < 8)
            def _():
                pltpu.make_async_copy(
                    hp.at[srcb.at[jnp.minimum(j + 2, 7)]], rows_a,
                    sem_a).start()

            process(base + _MCH, rows_b)
            return c2

        lax.fori_loop(0, 4, pair, 0)
        return c

    lax.fori_loop(g0, g1, group, 0)
    pltpu.sync_copy(
        acc, out.at[pl.ds(node_lo * f, _NODES_PER_SUB * f)])


@functools.lru_cache(maxsize=None)
def _segmax_call(f):
    return pl.kernel(
        functools.partial(_segmax_body, f),
        out_type=jax.ShapeDtypeStruct((_NODE_ROWS * f,), jnp.float32),
        mesh=plsc.VectorSubcoreMesh(core_axis_name="c", subcore_axis_name="s"),
        scratch_types=[
            pltpu.VMEM((8, _MCH), jnp.int32),
            pltpu.VMEM((_MCH + 16,), jnp.int32),
            pltpu.VMEM((_MCH, f), jnp.float32),
            pltpu.VMEM((_MCH, f), jnp.float32),
            pltpu.VMEM((48,), jnp.int32),
            pltpu.VMEM((_NODES_PER_SUB * f,), jnp.float32),
            pltpu.SemaphoreType.DMA,
            pltpu.SemaphoreType.DMA,
        ],
    )


def _sc_segmax(hp, srcg, dstf, offs48):
    f = hp.shape[1]
    out = _segmax_call(f)(hp, srcg, dstf, offs48)
    return out.reshape(_NODE_ROWS, f)[:_N]


def _sc_segsum(x, srcp, dstp):
    halves = x.shape[1] == 256
    z = jnp.zeros((_ACC_ROWS, 128), jnp.float32)
    if halves:
        out0, out1 = _segsum_call(True)(x[:, :128], x[:, 128:], srcp, dstp, z)
    else:
        out0, out1 = _segsum_call(False)(x, x, srcp, dstp, z)
    return out0, out1, halves


def _mm_relu_body(x_ref, w_ref, b_ref, o_ref):
    o_ref[...] = jnp.maximum(
        jnp.dot(x_ref[...], w_ref[...], preferred_element_type=jnp.float32)
        + b_ref[...], 0.0)


def _pool_pre(x, wp, bp):
    n, fi = x.shape
    return pl.pallas_call(
        _mm_relu_body,
        grid=(n // _ROWB,),
        in_specs=[
            pl.BlockSpec((_ROWB, fi), lambda i: (i, 0)),
            pl.BlockSpec((fi, fi), lambda i: (0, 0)),
            pl.BlockSpec((1, fi), lambda i: (0, 0)),
        ],
        out_specs=pl.BlockSpec((_ROWB, fi), lambda i: (i, 0)),
        out_shape=jax.ShapeDtypeStruct((n, fi), jnp.float32),
    )(x, wp, bp.reshape(1, -1))


def _combine_body(stats, halves, x_ref, s0_ref, s1_ref, m_ref, a1_ref, a2_ref,
                  wx_ref, wsn_ref, wgn_ref, wpn_ref, b_ref, o_ref,
                  st_ref=None):
    xb = x_ref[...]
    if halves:
        sb = jnp.concatenate([s0_ref[...], s1_ref[...]], axis=1)
    else:
        sb = s0_ref[...] + s1_ref[...]
    out = jnp.dot(xb, wx_ref[...], preferred_element_type=jnp.float32)
    out += jnp.dot(a1_ref[...] * sb, wsn_ref[...],
                   preferred_element_type=jnp.float32)
    out += jnp.dot(a2_ref[...] * (sb + xb), wgn_ref[...],
                   preferred_element_type=jnp.float32)
    out += jnp.dot(m_ref[...], wpn_ref[...], preferred_element_type=jnp.float32)
    out += b_ref[...]
    if stats:
        o_ref[...] = out

        @pl.when(pl.program_id(0) == 0)
        def _():
            st_ref[...] = jnp.zeros_like(st_ref)

        st_ref[...] += jnp.concatenate(
            [jnp.sum(out, axis=0, keepdims=True),
             jnp.sum(out * out, axis=0, keepdims=True)], axis=0)
    else:
        mx = jnp.max(out, axis=1, keepdims=True)
        lse = jnp.log(jnp.sum(jnp.exp(out - mx), axis=1, keepdims=True)) + mx
        o_ref[...] = out - lse


def _combine(x, s0, s1, halves, m, a1, a2, wx, wsn, wgn, wpn, b, stats):
    n, fi = x.shape
    fo = wx.shape[1]
    in_specs = [
        pl.BlockSpec((_ROWB, fi), lambda i: (i, 0)),
        pl.BlockSpec((_ROWB, 128), lambda i: (i, 0)),
        pl.BlockSpec((_ROWB, 128), lambda i: (i, 0)),
        pl.BlockSpec((_ROWB, fi), lambda i: (i, 0)),
        pl.BlockSpec((_ROWB, 1), lambda i: (i, 0)),
        pl.BlockSpec((_ROWB, 1), lambda i: (i, 0)),
        pl.BlockSpec((fi, fo), lambda i: (0, 0)),
        pl.BlockSpec((fi, fo), lambda i: (0, 0)),
        pl.BlockSpec((fi, fo), lambda i: (0, 0)),
        pl.BlockSpec((fi, fo), lambda i: (0, 0)),
        pl.BlockSpec((1, fo), lambda i: (0, 0)),
    ]
    if stats:
        out_specs = [pl.BlockSpec((_ROWB, fo), lambda i: (i, 0)),
                     pl.BlockSpec((2, fo), lambda i: (0, 0))]
        out_shape = [jax.ShapeDtypeStruct((n, fo), jnp.float32),
                     jax.ShapeDtypeStruct((2, fo), jnp.float32)]
    else:
        out_specs = pl.BlockSpec((_ROWB, fo), lambda i: (i, 0))
        out_shape = jax.ShapeDtypeStruct((n, fo), jnp.float32)
    return pl.pallas_call(
        functools.partial(_combine_body, stats, halves),
        grid=(n // _ROWB,),
        in_specs=in_specs,
        out_specs=out_specs,
        out_shape=out_shape,
    )(x, s0, s1, m, a1, a2, wx, wsn, wgn, wpn, b.reshape(1, -1))


def _bn_relu_body(h_ref, st_ref, g_ref, b_ref, o_ref):
    s1 = st_ref[0:1, :]
    s2 = st_ref[1:2, :]
    mean = s1 / _N
    var = s2 / _N - mean * mean
    inv = jax.lax.rsqrt(var + _EPS)
    o_ref[...] = jnp.maximum(
        g_ref[...] * (h_ref[...] - mean) * inv + b_ref[...], 0.0)


def _bn_relu(h, st, g, b):
    n, f = h.shape
    return pl.pallas_call(
        _bn_relu_body,
        grid=(n // _ROWB,),
        in_specs=[
            pl.BlockSpec((_ROWB, f), lambda i: (i, 0)),
            pl.BlockSpec((2, f), lambda i: (0, 0)),
            pl.BlockSpec((1, f), lambda i: (0, 0)),
            pl.BlockSpec((1, f), lambda i: (0, 0)),
        ],
        out_specs=pl.BlockSpec((_ROWB, f), lambda i: (i, 0)),
        out_shape=jax.ShapeDtypeStruct((n, f), jnp.float32),
    )(h, st, g.reshape(1, -1), b.reshape(1, -1))


def _layer(x, srcp, dstp, srcg, dstf, offs48, a1, a2, p, wts, stats):
    s0, s1, halves = _sc_segsum(x, srcp, dstp)
    hp = _pool_pre(x, p['pool']['Wp'], p['pool']['bp'])
    m = _sc_segmax(hp, srcg, dstf, offs48)
    w0, w1, w2 = wts[0], wts[1], wts[2]
    wx = w0 * p['mean']['Ws'] + w1 * p['pool']['Ws']
    wsn = w0 * p['mean']['Wn']
    wgn = w2 * p['gcn']['Wn']
    wpn = w1 * p['pool']['Wn']
    b = w0 * p['mean']['b'] + w1 * p['pool']['b'] + w2 * p['gcn']['b']
    return _combine(x, s0, s1, halves, m, a1, a2, wx, wsn, wgn, wpn, b, stats)


def kernel(x, edge_index, params):
    src = edge_index[0]
    dst = edge_index[1]
    srcp = jnp.concatenate(
        [src, jnp.zeros((_E_PAD - _E,), jnp.int32)]).reshape(-1, _CHUNK)
    dstp = jnp.concatenate(
        [dst, jnp.full((_E_PAD - _E,), _TRASH, jnp.int32)]).reshape(-1, _CHUNK)

    dsts, srcs = lax.sort_key_val(dst, src)
    pad = _ES_PAD - _E
    dstf = jnp.concatenate(
        [dsts, jnp.full((pad,), jnp.int32(0x3FFFFFFF))])
    srcg = jnp.concatenate([srcs, jnp.zeros((pad,), jnp.int32)]
                           ).reshape(-1, _MCH)
    bounds = jnp.arange(0, _NODE_ROWS + 1, _NODES_PER_SUB, dtype=jnp.int32)
    offs33 = jnp.searchsorted(dsts, bounds).astype(jnp.int32)
    offs48 = jnp.concatenate([offs33, jnp.zeros((15,), jnp.int32)])
    offsn = jnp.searchsorted(dsts, jnp.arange(_N + 1, dtype=jnp.int32))
    deg = (offsn[1:] - offsn[:-1]).astype(jnp.float32)
    a1 = (1.0 / jnp.maximum(deg, 1.0)).reshape(-1, 1)
    a2 = (1.0 / (deg + 1.0)).reshape(-1, 1)

    w1 = jax.nn.softmax(params['c1']['w'])
    w3 = jax.nn.softmax(params['c3']['w'])
    w5 = jax.nn.softmax(params['c5']['w'])

    h, st = _layer(x, srcp, dstp, srcg, dstf, offs48, a1, a2,
                   params['c1'], w1, True)
    h = _bn_relu(h, st, params['bn2']['g'], params['bn2']['b'])
    h, st = _layer(h, srcp, dstp, srcg, dstf, offs48, a1, a2,
                   params['c3'], w3, True)
    h = _bn_relu(h, st, params['bn4']['g'], params['bn4']['b'])
    return _layer(h, srcp, dstp, srcg, dstf, offs48, a1, a2,
                  params['c5'], w5, False)
